# Initial kernel scaffold; baseline (speedup 1.0000x reference)
import jax
import jax.numpy as jnp
from jax.experimental import pallas as pl


def kernel(input_ids, word_emb, pos_emb, tok_emb, ln_w, ln_b):
    raise NotImplementedError("write your pallas kernel here")


# speedup vs baseline: 1.2594x; 1.2594x over previous
import functools

import jax
import jax.numpy as jnp
from jax import lax
from jax.experimental import pallas as pl
from jax.experimental.pallas import tpu as pltpu
from jax.experimental.pallas import tpu_sc as plsc

HIDDEN = 768
PAD_IDX = 1
EPS = 1e-05

_NC = 2
_NS = 16
_NW = _NC * _NS


def _posid_body(ids_ref, out_ref):
    ids = ids_ref[...]
    mask = (ids != PAD_IDX).astype(jnp.int32)
    c = mask
    n = ids.shape[1]
    k = 1
    while k < n:
        zeros = jnp.zeros((ids.shape[0], k), dtype=jnp.int32)
        c = c + jnp.concatenate([zeros, c[:, : n - k]], axis=1)
        k *= 2
    out_ref[...] = c * mask + PAD_IDX


def _position_ids(input_ids):
    return pl.pallas_call(
        _posid_body,
        out_shape=jax.ShapeDtypeStruct(input_ids.shape, jnp.int32),
    )(input_ids)


def _sc_gather_sum(word_emb, pos_emb, ids_flat, pid_flat):
    tok = ids_flat.shape[0]
    tpw = tok // _NW
    ch = 64
    nchunk = tpw // ch
    mesh = plsc.VectorSubcoreMesh(core_axis_name="c", subcore_axis_name="s")

    @functools.partial(
        pl.kernel,
        out_type=jax.ShapeDtypeStruct((tok, HIDDEN), jnp.float32),
        mesh=mesh,
        scratch_types=[
            pltpu.VMEM((tpw,), jnp.int32),
            pltpu.VMEM((tpw,), jnp.int32),
            pltpu.VMEM((ch, HIDDEN), jnp.float32),
            pltpu.VMEM((ch, HIDDEN), jnp.float32),
            pltpu.SemaphoreType.DMA,
        ],
    )
    def k(word_hbm, pos_hbm, ids_hbm, pid_hbm, out_hbm, ids_v, pid_v, wbuf,
          pbuf, sem):
        wid = lax.axis_index("s") * _NC + lax.axis_index("c")
        base = wid * tpw
        pltpu.sync_copy(ids_hbm.at[pl.ds(base, tpw)], ids_v)
        pltpu.sync_copy(pid_hbm.at[pl.ds(base, tpw)], pid_v)
        for g in range(nchunk):
            cw = pltpu.async_copy(
                word_hbm.at[ids_v.at[pl.ds(g * ch, ch)]], wbuf, sem)
            cp = pltpu.async_copy(
                pos_hbm.at[pid_v.at[pl.ds(g * ch, ch)]], pbuf, sem)
            cw.wait()
            cp.wait()

            @pl.loop(0, ch)
            def _row(r):
                @pl.loop(0, HIDDEN, step=16)
                def _col(c):
                    sl = (r, pl.ds(c, 16))
                    wbuf[sl] = wbuf[sl] + pbuf[sl]

            pltpu.sync_copy(wbuf, out_hbm.at[pl.ds(base + g * ch, ch)])

    return k(word_emb, pos_emb, ids_flat, pid_flat)


def _ln_body(x_ref, tok_ref, w_ref, b_ref, o_ref):
    x = x_ref[...] + tok_ref[...]
    mean = jnp.mean(x, axis=-1, keepdims=True)
    xc = x - mean
    var = jnp.mean(xc * xc, axis=-1, keepdims=True)
    o_ref[...] = xc * lax.rsqrt(var + EPS) * w_ref[...] + b_ref[...]


def _ln(summed, tok_row, ln_w, ln_b):
    tok = summed.shape[0]
    blk = 512
    return pl.pallas_call(
        _ln_body,
        grid=(tok // blk,),
        in_specs=[
            pl.BlockSpec((blk, HIDDEN), lambda i: (i, 0)),
            pl.BlockSpec((1, HIDDEN), lambda i: (0, 0)),
            pl.BlockSpec((1, HIDDEN), lambda i: (0, 0)),
            pl.BlockSpec((1, HIDDEN), lambda i: (0, 0)),
        ],
        out_specs=pl.BlockSpec((blk, HIDDEN), lambda i: (i, 0)),
        out_shape=jax.ShapeDtypeStruct((tok, HIDDEN), jnp.float32),
    )(summed, tok_row, ln_w, ln_b)


def kernel(input_ids, word_emb, pos_emb, tok_emb, ln_w, ln_b):
    b, s = input_ids.shape
    position_ids = _position_ids(input_ids)
    summed = _sc_gather_sum(word_emb, pos_emb,
                            input_ids.reshape(-1),
                            position_ids.reshape(-1))
    out = _ln(summed, tok_emb[0:1], ln_w.reshape(1, HIDDEN),
              ln_b.reshape(1, HIDDEN))
    return out.reshape(b, s, HIDDEN), position_ids


# speedup vs baseline: 2.1606x; 1.7156x over previous
import functools

import jax
import jax.numpy as jnp
from jax import lax
from jax.experimental import pallas as pl
from jax.experimental.pallas import tpu as pltpu
from jax.experimental.pallas import tpu_sc as plsc

HIDDEN = 768
PAD_IDX = 1
EPS = 1e-05

_NC = 2
_NS = 16
_NW = _NC * _NS


def _posid_body(ids_ref, out_ref):
    ids = ids_ref[...]
    mask = (ids != PAD_IDX).astype(jnp.int32)
    c = mask
    n = ids.shape[1]
    k = 1
    while k < n:
        zeros = jnp.zeros((ids.shape[0], k), dtype=jnp.int32)
        c = c + jnp.concatenate([zeros, c[:, : n - k]], axis=1)
        k *= 2
    out_ref[...] = c * mask + PAD_IDX


def _position_ids(input_ids):
    return pl.pallas_call(
        _posid_body,
        out_shape=jax.ShapeDtypeStruct(input_ids.shape, jnp.int32),
    )(input_ids)


def _sc_gather_sum(word_emb, pos_emb, ids_flat, pid_flat):
    tok = ids_flat.shape[0]
    tpw = tok // _NW
    ch = 16
    nchunk = tpw // ch
    mesh = plsc.VectorSubcoreMesh(core_axis_name="c", subcore_axis_name="s")
    buf = pltpu.VMEM((ch, HIDDEN), jnp.float32)

    @functools.partial(
        pl.kernel,
        out_type=jax.ShapeDtypeStruct((tok, HIDDEN), jnp.float32),
        mesh=mesh,
        scratch_types=[
            pltpu.VMEM((tpw,), jnp.int32),
            pltpu.VMEM((tpw,), jnp.int32),
            buf, buf, buf, buf, buf, buf,
            pltpu.SemaphoreType.DMA, pltpu.SemaphoreType.DMA,
            pltpu.SemaphoreType.DMA, pltpu.SemaphoreType.DMA,
            pltpu.SemaphoreType.DMA, pltpu.SemaphoreType.DMA,
        ],
    )
    def k(word_hbm, pos_hbm, ids_hbm, pid_hbm, out_hbm, ids_v, pid_v,
          w0, w1, p0, p1, o0, o1, ws0, ws1, ps0, ps1, os0, os1):
        wb, pb, ob = [w0, w1], [p0, p1], [o0, o1]
        wsem, psem, osem = [ws0, ws1], [ps0, ps1], [os0, os1]
        wid = lax.axis_index("s") * _NC + lax.axis_index("c")
        base = wid * tpw
        pltpu.sync_copy(ids_hbm.at[pl.ds(base, tpw)], ids_v)
        pltpu.sync_copy(pid_hbm.at[pl.ds(base, tpw)], pid_v)

        def issue_gathers(i, b):
            pltpu.async_copy(
                word_hbm.at[ids_v.at[pl.ds(i * ch, ch)]], wb[b], wsem[b])
            pltpu.async_copy(
                pos_hbm.at[pid_v.at[pl.ds(i * ch, ch)]], pb[b], psem[b])

        def wait_gathers(b):
            pltpu.make_async_copy(
                word_hbm.at[ids_v.at[pl.ds(0, ch)]], wb[b], wsem[b]).wait()
            pltpu.make_async_copy(
                pos_hbm.at[pid_v.at[pl.ds(0, ch)]], pb[b], psem[b]).wait()

        def wait_owrite(b):
            pltpu.make_async_copy(
                ob[b], out_hbm.at[pl.ds(base, ch)], osem[b]).wait()

        issue_gathers(0, 0)
        issue_gathers(1, 1)

        @pl.loop(0, nchunk, step=2)
        def _pair(g):
            for b in range(2):
                wait_gathers(b)

                @pl.when(g >= 2 - b)
                def _():
                    wait_owrite(b)

                @pl.loop(0, ch)
                def _row(r):
                    for c in range(0, HIDDEN, 16):
                        sl = (r, pl.ds(c, 16))
                        ob[b][sl] = wb[b][sl] + pb[b][sl]

                pltpu.async_copy(
                    ob[b], out_hbm.at[pl.ds(base + (g + b) * ch, ch)],
                    osem[b])

                @pl.when(g < nchunk - 2 - b)
                def _():
                    issue_gathers(g + b + 2, b)

        wait_owrite(0)
        wait_owrite(1)

    return k(word_emb, pos_emb, ids_flat, pid_flat)


def _ln_body(x_ref, tok_ref, w_ref, b_ref, o_ref):
    x = x_ref[...] + tok_ref[...]
    mean = jnp.mean(x, axis=-1, keepdims=True)
    xc = x - mean
    var = jnp.mean(xc * xc, axis=-1, keepdims=True)
    o_ref[...] = xc * lax.rsqrt(var + EPS) * w_ref[...] + b_ref[...]


def _ln(summed, tok_row, ln_w, ln_b):
    tok = summed.shape[0]
    blk = 512
    return pl.pallas_call(
        _ln_body,
        grid=(tok // blk,),
        in_specs=[
            pl.BlockSpec((blk, HIDDEN), lambda i: (i, 0)),
            pl.BlockSpec((1, HIDDEN), lambda i: (0, 0)),
            pl.BlockSpec((1, HIDDEN), lambda i: (0, 0)),
            pl.BlockSpec((1, HIDDEN), lambda i: (0, 0)),
        ],
        out_specs=pl.BlockSpec((blk, HIDDEN), lambda i: (i, 0)),
        out_shape=jax.ShapeDtypeStruct((tok, HIDDEN), jnp.float32),
    )(summed, tok_row, ln_w, ln_b)


def kernel(input_ids, word_emb, pos_emb, tok_emb, ln_w, ln_b):
    b, s = input_ids.shape
    position_ids = _position_ids(input_ids)
    summed = _sc_gather_sum(word_emb, pos_emb,
                            input_ids.reshape(-1),
                            position_ids.reshape(-1))
    out = _ln(summed, tok_emb[0:1], ln_w.reshape(1, HIDDEN),
              ln_b.reshape(1, HIDDEN))
    return out.reshape(b, s, HIDDEN), position_ids
